# baseline (speedup 1.0000x reference)
import functools

import jax
import jax.numpy as jnp
from jax import lax
from jax.experimental import pallas as pl
from jax.experimental.pallas import tpu as pltpu
from jax.experimental.pallas import tpu_sc as plsc

N = 1_000_000
B = 100_000
NC = 2
NS = 16
NW = NC * NS
L = 16

C_BASE = 31_248
C_TAIL = N - (NW - 1) * C_BASE
C_PAD = 31_328
CHUNK = 10_000
NCHUNKS = B // CHUNK
NBUF = 8
VREGS = CHUNK // L
UNROLL = 5
TAIL_VREGS = VREGS % UNROLL
ZUNROLL = 11


def _make_kernel():
  mesh = plsc.VectorSubcoreMesh(core_axis_name="c", subcore_axis_name="s")

  @functools.partial(
      pl.kernel,
      out_type=jax.ShapeDtypeStruct((N,), jnp.int32),
      mesh=mesh,
      scratch_types=[
          [pltpu.VMEM((CHUNK,), jnp.int32)] * NBUF,
          pltpu.VMEM((C_PAD,), jnp.int32),
          [pltpu.SemaphoreType.DMA] * NBUF,
      ],
      compiler_params=pltpu.CompilerParams(needs_layout_passes=False),
  )
  def idx_map_kernel(idx_hbm, out_hbm, idx_bufs, out_v, sems):
    wid = lax.axis_index("s") * NC + lax.axis_index("c")
    lo = wid * C_BASE
    is_last = wid == NW - 1
    size = jnp.where(is_last, jnp.int32(C_TAIL), jnp.int32(C_BASE)).astype(
        jnp.uint32)
    iota = lax.iota(jnp.int32, L)
    zvec = jnp.zeros((L,), jnp.int32)

    def start_copy(k):
      pltpu.async_copy(
          idx_hbm.at[pl.ds(k * CHUNK, CHUNK)], idx_bufs[k % NBUF],
          sems[k % NBUF])

    def wait_copy(k):
      pltpu.make_async_copy(
          idx_hbm.at[pl.ds(k * CHUNK, CHUNK)], idx_bufs[k % NBUF],
          sems[k % NBUF]).wait()

    for k in range(NBUF - 1):
      start_copy(k)

    def zero_body(g, carry):
      for j in range(ZUNROLL):
        out_v[pl.ds((g * ZUNROLL + j) * L, L)] = zvec
      return carry

    lax.fori_loop(0, C_PAD // (ZUNROLL * L), zero_body, 0, unroll=False)

    def group(buf, vals, g, nv):
      locs, vgroup, masks = [], [], []
      for j in range(nv):
        ids = buf[pl.ds((g * UNROLL + j) * L, L)]
        loc = ids - lo
        m = loc.astype(jnp.uint32) < size
        locs.append(loc)
        vgroup.append(vals)
        masks.append(m)
        vals = vals + L
      for j in range(nv):
        plsc.store_scatter(out_v, [locs[j]], vgroup[j], mask=masks[j])
      return vals

    def process(k):
      buf = idx_bufs[k % NBUF]
      base = k * CHUNK

      def vbody(g, vals):
        return group(buf, vals, g, UNROLL)

      vals = lax.fori_loop(
          0, VREGS // UNROLL, vbody, iota + base, unroll=False)
      if TAIL_VREGS:
        group(buf, vals, VREGS // UNROLL, TAIL_VREGS)

    for k in range(NCHUNKS):
      wait_copy(k)
      if k + NBUF - 1 < NCHUNKS:
        start_copy(k + NBUF - 1)
      process(k)

    pltpu.sync_copy(out_v.at[pl.ds(0, C_BASE)], out_hbm.at[pl.ds(lo, C_BASE)])

    @pl.when(is_last)
    def _():
      pltpu.sync_copy(
          out_v.at[pl.ds(C_BASE, C_TAIL - C_BASE)],
          out_hbm.at[pl.ds(N - (C_TAIL - C_BASE), C_TAIL - C_BASE)],
      )

  return idx_map_kernel


_KERNEL = _make_kernel()


def kernel(X, row_missing_idx):
  del X
  return _KERNEL(row_missing_idx.astype(jnp.int32))


# speedup vs baseline: 1.0610x; 1.0610x over previous
import functools

import jax
import jax.numpy as jnp
from jax import lax
from jax.experimental import pallas as pl
from jax.experimental.pallas import tpu as pltpu
from jax.experimental.pallas import tpu_sc as plsc

N = 1_000_000
B = 100_000
NC = 2
NS = 16
NW = NC * NS
L = 16

C_BASE = 31_248
C_TAIL = N - (NW - 1) * C_BASE
C_PAD = 31_328
CHUNK = 10_000
NCHUNKS = B // CHUNK
NBUF = 4
VREGS = CHUNK // L
UNROLL = 5
TAIL_VREGS = VREGS % UNROLL
ZUNROLL = 11


def _make_kernel():
  mesh = plsc.VectorSubcoreMesh(core_axis_name="c", subcore_axis_name="s")

  @functools.partial(
      pl.kernel,
      out_type=jax.ShapeDtypeStruct((N,), jnp.int32),
      mesh=mesh,
      scratch_types=[
          [pltpu.VMEM((CHUNK,), jnp.int32)] * NBUF,
          pltpu.VMEM((C_PAD,), jnp.int32),
          [pltpu.SemaphoreType.DMA] * NBUF,
      ],
      compiler_params=pltpu.CompilerParams(needs_layout_passes=False),
  )
  def idx_map_kernel(idx_hbm, out_hbm, idx_bufs, out_v, sems):
    wid = lax.axis_index("s") * NC + lax.axis_index("c")
    lo = wid * C_BASE
    is_last = wid == NW - 1
    size = jnp.where(is_last, jnp.int32(C_TAIL), jnp.int32(C_BASE)).astype(
        jnp.uint32)
    iota = lax.iota(jnp.int32, L)
    zvec = jnp.zeros((L,), jnp.int32)

    def start_copy(k):
      pltpu.async_copy(
          idx_hbm.at[pl.ds(k * CHUNK, CHUNK)], idx_bufs[k % NBUF],
          sems[k % NBUF])

    def wait_copy(k):
      pltpu.make_async_copy(
          idx_hbm.at[pl.ds(k * CHUNK, CHUNK)], idx_bufs[k % NBUF],
          sems[k % NBUF]).wait()

    for k in range(NBUF - 1):
      start_copy(k)

    def zero_body(g, carry):
      for j in range(ZUNROLL):
        out_v[pl.ds((g * ZUNROLL + j) * L, L)] = zvec
      return carry

    lax.fori_loop(0, C_PAD // (ZUNROLL * L), zero_body, 0, unroll=False)

    def group(buf, vals, g, nv):
      locs, vgroup, masks = [], [], []
      for j in range(nv):
        ids = buf[pl.ds((g * UNROLL + j) * L, L)]
        loc = ids - lo
        m = loc.astype(jnp.uint32) < size
        locs.append(loc)
        vgroup.append(vals)
        masks.append(m)
        vals = vals + L
      for j in range(nv):
        plsc.store_scatter(out_v, [locs[j]], vgroup[j], mask=masks[j])
      return vals

    def process(k):
      buf = idx_bufs[k % NBUF]
      base = k * CHUNK

      def vbody(g, vals):
        return group(buf, vals, g, UNROLL)

      vals = lax.fori_loop(
          0, VREGS // UNROLL, vbody, iota + base, unroll=False)
      if TAIL_VREGS:
        group(buf, vals, VREGS // UNROLL, TAIL_VREGS)

    for k in range(NCHUNKS):
      wait_copy(k)
      if k + NBUF - 1 < NCHUNKS:
        start_copy(k + NBUF - 1)
      process(k)

    pltpu.sync_copy(out_v.at[pl.ds(0, C_BASE)], out_hbm.at[pl.ds(lo, C_BASE)])

    @pl.when(is_last)
    def _():
      pltpu.sync_copy(
          out_v.at[pl.ds(C_BASE, C_TAIL - C_BASE)],
          out_hbm.at[pl.ds(N - (C_TAIL - C_BASE), C_TAIL - C_BASE)],
      )

  return idx_map_kernel


_KERNEL = _make_kernel()


def kernel(X, row_missing_idx):
  del X
  return _KERNEL(row_missing_idx.astype(jnp.int32))
